# Initial kernel scaffold; baseline (speedup 1.0000x reference)
import jax
import jax.numpy as jnp
from jax.experimental import pallas as pl


def kernel(feature, label, centers):
    raise NotImplementedError("write your pallas kernel here")


# speedup vs baseline: 1.0016x; 1.0016x over previous
import functools

import jax
import jax.numpy as jnp
from jax import lax
from jax.experimental import pallas as pl
from jax.experimental.pallas import tpu as pltpu
from jax.experimental.pallas import tpu_sc as plsc

B = 16384
D = 128
NW = 32
B_PER_W = B // NW
CHUNK = 128
NCHUNK = B_PER_W // CHUNK
GROUPS = D // 16


def _sc_partials(feature, label, centers):
    mesh = plsc.VectorSubcoreMesh(core_axis_name="c", subcore_axis_name="s")

    @functools.partial(
        pl.kernel,
        mesh=mesh,
        out_type=jax.ShapeDtypeStruct((NW * 16,), jnp.float32),
        scratch_types=[
            pltpu.VMEM((NCHUNK, CHUNK), jnp.int32),
            pltpu.VMEM((CHUNK, D), jnp.float32),
            pltpu.VMEM((CHUNK, D), jnp.float32),
            pltpu.VMEM((16,), jnp.float32),
            pltpu.SemaphoreType.DMA,
        ],
    )
    def k(feat_hbm, lab_hbm, cent_hbm, out_hbm, idx_v, cent_v, feat_v, res_v, sem):
        wid = lax.axis_index("s") * 2 + lax.axis_index("c")
        base = wid * B_PER_W
        for c in range(NCHUNK):
            pltpu.sync_copy(lab_hbm.at[pl.ds(base + c * CHUNK, CHUNK)], idx_v.at[c])

        acc = tuple(jnp.zeros((16,), jnp.float32) for _ in range(GROUPS))
        for c in range(NCHUNK):
            pltpu.async_copy(cent_hbm.at[idx_v.at[c]], cent_v, sem).wait()
            pltpu.sync_copy(feat_hbm.at[pl.ds(base + c * CHUNK, CHUNK)], feat_v)

            def body(r, a):
                new = []
                for g in range(GROUPS):
                    f = feat_v[r, pl.ds(g * 16, 16)]
                    ce = cent_v[r, pl.ds(g * 16, 16)]
                    d_ = f - ce
                    new.append(a[g] + d_ * d_)
                return tuple(new)

            acc = lax.fori_loop(0, CHUNK, body, acc)

        total = acc[0]
        for g in range(1, GROUPS):
            total = total + acc[g]
        res_v[...] = total
        pltpu.sync_copy(res_v, out_hbm.at[pl.ds(wid * 16, 16)])

    return k(feature, label, centers)


def _tc_sum(partials):
    x = partials.reshape(4, 128)

    def body(x_ref, o_ref):
        o_ref[0, 0] = jnp.sum(x_ref[...]) * 0.5

    out = pl.pallas_call(
        body,
        out_shape=jax.ShapeDtypeStruct((1, 1), jnp.float32),
        out_specs=pl.BlockSpec(memory_space=pltpu.SMEM),
    )(x)
    return out[0, 0]


@jax.jit
def kernel(feature, label, centers):
    partials = _sc_partials(feature, label, centers)
    return _tc_sum(partials)


# speedup vs baseline: 1.1855x; 1.1835x over previous
import functools

import jax
import jax.numpy as jnp
from jax import lax
from jax.experimental import pallas as pl
from jax.experimental.pallas import tpu as pltpu
from jax.experimental.pallas import tpu_sc as plsc

B = 16384
D = 128
NW = 32
B_PER_W = B // NW
CHUNK = 128
NCHUNK = B_PER_W // CHUNK
GROUPS = D // 16


def _sc_partials(feature, label, centers):
    mesh = plsc.VectorSubcoreMesh(core_axis_name="c", subcore_axis_name="s")

    @functools.partial(
        pl.kernel,
        mesh=mesh,
        out_type=jax.ShapeDtypeStruct((NW * 16,), jnp.float32),
        scratch_types=[
            pltpu.VMEM((NCHUNK, CHUNK), jnp.int32),
            pltpu.VMEM((2, CHUNK, D), jnp.float32),
            pltpu.VMEM((2, CHUNK, D), jnp.float32),
            pltpu.VMEM((16,), jnp.float32),
            pltpu.SemaphoreType.DMA,
            pltpu.SemaphoreType.DMA,
            pltpu.SemaphoreType.DMA,
            pltpu.SemaphoreType.DMA,
        ],
    )
    def k(feat_hbm, lab_hbm, cent_hbm, out_hbm, idx_v, cent_v, feat_v, res_v,
          sc0, sc1, sf0, sf1):
        wid = lax.axis_index("s") * 2 + lax.axis_index("c")
        base = wid * B_PER_W
        for c in range(NCHUNK):
            pltpu.sync_copy(lab_hbm.at[pl.ds(base + c * CHUNK, CHUNK)], idx_v.at[c])

        sems_c = (sc0, sc1)
        sems_f = (sf0, sf1)

        def issue(c):
            s = c % 2
            hc = pltpu.async_copy(cent_hbm.at[idx_v.at[c]], cent_v.at[s], sems_c[s])
            hf = pltpu.async_copy(feat_hbm.at[pl.ds(base + c * CHUNK, CHUNK)],
                                  feat_v.at[s], sems_f[s])
            return hc, hf

        pend = [None] * NCHUNK
        pend[0] = issue(0)

        acc = tuple(jnp.zeros((16,), jnp.float32) for _ in range(GROUPS))
        for c in range(NCHUNK):
            if c + 1 < NCHUNK:
                pend[c + 1] = issue(c + 1)
            hc, hf = pend[c]
            hc.wait()
            hf.wait()
            s = c % 2

            def body(r, a, s=s):
                new = []
                for g in range(GROUPS):
                    f = feat_v[s, r, pl.ds(g * 16, 16)]
                    ce = cent_v[s, r, pl.ds(g * 16, 16)]
                    d_ = f - ce
                    new.append(a[g] + d_ * d_)
                return tuple(new)

            acc = lax.fori_loop(0, CHUNK, body, acc)

        total = acc[0]
        for g in range(1, GROUPS):
            total = total + acc[g]
        res_v[...] = total
        pltpu.sync_copy(res_v, out_hbm.at[pl.ds(wid * 16, 16)])

    return k(feature, label, centers)


def _tc_sum(partials):
    x = partials.reshape(4, 128)

    def body(x_ref, o_ref):
        o_ref[0, 0] = jnp.sum(x_ref[...]) * 0.5

    out = pl.pallas_call(
        body,
        out_shape=jax.ShapeDtypeStruct((1, 1), jnp.float32),
        out_specs=pl.BlockSpec(memory_space=pltpu.SMEM),
    )(x)
    return out[0, 0]


@jax.jit
def kernel(feature, label, centers):
    partials = _sc_partials(feature, label, centers)
    return _tc_sum(partials)


# speedup vs baseline: 1.1860x; 1.0005x over previous
import functools

import jax
import jax.numpy as jnp
from jax import lax
from jax.experimental import pallas as pl
from jax.experimental.pallas import tpu as pltpu
from jax.experimental.pallas import tpu_sc as plsc

B = 16384
D = 128
NW = 32
B_PER_W = B // NW
CHUNK = 128
NCHUNK = B_PER_W // CHUNK
GROUPS = D // 16


def _sc_partials(feature, label, centers):
    mesh = plsc.VectorSubcoreMesh(core_axis_name="c", subcore_axis_name="s")

    @functools.partial(
        pl.kernel,
        mesh=mesh,
        out_type=jax.ShapeDtypeStruct((NW * 16,), jnp.float32),
        scratch_types=[
            pltpu.VMEM((NCHUNK, CHUNK), jnp.int32),
            pltpu.VMEM((2, CHUNK, D), jnp.float32),
            pltpu.VMEM((2, CHUNK, D), jnp.float32),
            pltpu.VMEM((16,), jnp.float32),
            pltpu.SemaphoreType.DMA,
            pltpu.SemaphoreType.DMA,
            pltpu.SemaphoreType.DMA,
            pltpu.SemaphoreType.DMA,
        ],
    )
    def k(feat_hbm, lab_hbm, cent_hbm, out_hbm, idx_v, cent_v, feat_v, res_v,
          sc0, sc1, sf0, sf1):
        wid = lax.axis_index("s") * 2 + lax.axis_index("c")
        base = wid * B_PER_W
        for c in range(NCHUNK):
            pltpu.sync_copy(lab_hbm.at[pl.ds(base + c * CHUNK, CHUNK)], idx_v.at[c])

        sems_c = (sc0, sc1)
        sems_f = (sf0, sf1)

        def issue(c):
            s = c % 2
            hc = pltpu.async_copy(cent_hbm.at[idx_v.at[c]], cent_v.at[s], sems_c[s])
            hf = pltpu.async_copy(feat_hbm.at[pl.ds(base + c * CHUNK, CHUNK)],
                                  feat_v.at[s], sems_f[s])
            return hc, hf

        pend = [None] * NCHUNK
        pend[0] = issue(0)

        acc = tuple(jnp.zeros((16,), jnp.float32) for _ in range(GROUPS))
        for c in range(NCHUNK):
            if c + 1 < NCHUNK:
                pend[c + 1] = issue(c + 1)
            hc, hf = pend[c]
            hc.wait()
            hf.wait()
            s = c % 2

            @plsc.parallel_loop(0, CHUNK, step=1, unroll=4, carry=acc)
            def acc(r, a, s=s):
                new = []
                for g in range(GROUPS):
                    f = feat_v[s, r, pl.ds(g * 16, 16)]
                    ce = cent_v[s, r, pl.ds(g * 16, 16)]
                    d_ = f - ce
                    new.append(a[g] + d_ * d_)
                return tuple(new)

        total = acc[0]
        for g in range(1, GROUPS):
            total = total + acc[g]
        res_v[...] = total
        pltpu.sync_copy(res_v, out_hbm.at[pl.ds(wid * 16, 16)])

    return k(feature, label, centers)


def _tc_sum(partials):
    x = partials.reshape(4, 128)

    def body(x_ref, o_ref):
        o_ref[0, 0] = jnp.sum(x_ref[...]) * 0.5

    out = pl.pallas_call(
        body,
        out_shape=jax.ShapeDtypeStruct((1, 1), jnp.float32),
        out_specs=pl.BlockSpec(memory_space=pltpu.SMEM),
    )(x)
    return out[0, 0]


@jax.jit
def kernel(feature, label, centers):
    partials = _sc_partials(feature, label, centers)
    return _tc_sum(partials)


# speedup vs baseline: 1.2422x; 1.0474x over previous
import functools

import jax
import jax.numpy as jnp
from jax import lax
from jax.experimental import pallas as pl
from jax.experimental.pallas import tpu as pltpu
from jax.experimental.pallas import tpu_sc as plsc

B = 16384
D = 128
NW = 32
B_PER_W = B // NW
CHUNK = 128
NCHUNK = B_PER_W // CHUNK
GROUPS = D // 16


def _sc_partials(feature, label, centers):
    mesh = plsc.VectorSubcoreMesh(core_axis_name="c", subcore_axis_name="s")

    @functools.partial(
        pl.kernel,
        mesh=mesh,
        out_type=jax.ShapeDtypeStruct((NW * 16,), jnp.float32),
        scratch_types=[
            pltpu.VMEM((NCHUNK, CHUNK), jnp.int32),
            pltpu.VMEM((NCHUNK, CHUNK, D), jnp.float32),
            pltpu.VMEM((2, CHUNK, D), jnp.float32),
            pltpu.VMEM((16,), jnp.float32),
            pltpu.SemaphoreType.DMA,
            pltpu.SemaphoreType.DMA,
            pltpu.SemaphoreType.DMA,
            pltpu.SemaphoreType.DMA,
            pltpu.SemaphoreType.DMA,
            pltpu.SemaphoreType.DMA,
            pltpu.SemaphoreType.DMA,
        ],
    )
    def k(feat_hbm, lab_hbm, cent_hbm, out_hbm, idx_v, cent_v, feat_v, res_v,
          sidx, sc0, sc1, sc2, sc3, sf0, sf1):
        wid = lax.axis_index("s") * 2 + lax.axis_index("c")
        base = wid * B_PER_W
        sems_c = (sc0, sc1, sc2, sc3)
        sems_f = (sf0, sf1)

        hidx = [pltpu.async_copy(lab_hbm.at[pl.ds(base + c * CHUNK, CHUNK)],
                                 idx_v.at[c], sidx)
                for c in range(NCHUNK)]
        hf = [None] * NCHUNK
        for c in range(2):
            hf[c] = pltpu.async_copy(feat_hbm.at[pl.ds(base + c * CHUNK, CHUNK)],
                                     feat_v.at[c % 2], sems_f[c % 2])
        for h in hidx:
            h.wait()
        hc = [pltpu.async_copy(cent_hbm.at[idx_v.at[c]], cent_v.at[c], sems_c[c])
              for c in range(NCHUNK)]

        acc = tuple(jnp.zeros((16,), jnp.float32) for _ in range(GROUPS))
        for c in range(NCHUNK):
            hc[c].wait()
            hf[c].wait()

            @plsc.parallel_loop(0, CHUNK, step=1, unroll=4, carry=acc)
            def acc(r, a, c=c):
                new = []
                for g in range(GROUPS):
                    f = feat_v[c % 2, r, pl.ds(g * 16, 16)]
                    ce = cent_v[c, r, pl.ds(g * 16, 16)]
                    d_ = f - ce
                    new.append(a[g] + d_ * d_)
                return tuple(new)

            if c + 2 < NCHUNK:
                c2 = c + 2
                hf[c2] = pltpu.async_copy(
                    feat_hbm.at[pl.ds(base + c2 * CHUNK, CHUNK)],
                    feat_v.at[c2 % 2], sems_f[c2 % 2])

        total = acc[0]
        for g in range(1, GROUPS):
            total = total + acc[g]
        res_v[...] = total
        pltpu.sync_copy(res_v, out_hbm.at[pl.ds(wid * 16, 16)])

    return k(feature, label, centers)


def _tc_sum(partials):
    x = partials.reshape(4, 128)

    def body(x_ref, o_ref):
        o_ref[0, 0] = jnp.sum(x_ref[...]) * 0.5

    out = pl.pallas_call(
        body,
        out_shape=jax.ShapeDtypeStruct((1, 1), jnp.float32),
        out_specs=pl.BlockSpec(memory_space=pltpu.SMEM),
    )(x)
    return out[0, 0]


@jax.jit
def kernel(feature, label, centers):
    partials = _sc_partials(feature, label, centers)
    return _tc_sum(partials)
